# baseline (speedup 1.0000x reference)
import jax
import jax.numpy as jnp
from jax import lax
from jax.experimental import pallas as pl
from jax.experimental.pallas import tpu as pltpu
from jax.experimental.pallas import tpu_sc as plsc

_D = 64
_SCALE = 8.0
_W = 128
_L = 16
_NW = 32
_NB = 7812
_NK = 245


def _scale_table(tT_hbm, tail_hbm, out_hbm, in_v, st_v, rb_v, tv_v,
                 sem_in, sem_out):
    wid = lax.axis_index("s") * 2 + lax.axis_index("c")
    lanes = lax.iota(jnp.int32, _L)
    base = wid * (_NB // _NW) + jnp.minimum(wid, _NB % _NW)

    def band(k):
        return jnp.minimum(base + k, _NB - 1)

    def start_in(k, p):
        pltpu.async_copy(
            tT_hbm.at[:, pl.ds(band(k) * _W, _W)], in_v.at[p], sem_in.at[p])

    def wait_in(k, p):
        pltpu.make_async_copy(
            tT_hbm.at[:, pl.ds(band(k) * _W, _W)], in_v.at[p],
            sem_in.at[p]).wait()

    def start_out(k, p):
        pltpu.async_copy(
            rb_v.at[p], out_hbm.at[pl.ds(band(k) * _D, _D), :], sem_out.at[p])

    def wait_out(k, p):
        pltpu.make_async_copy(
            rb_v.at[p], out_hbm.at[pl.ds(band(k) * _D, _D), :],
            sem_out.at[p]).wait()

    v17 = lanes * 17

    def compute(p):
        @plsc.parallel_loop(0, _D, unroll=8)
        def _(d):
            for lg in range(_W // _L):
                st_v[pl.ds(lg * 1088 + d * 17, _L)] = (
                    in_v[p, d, pl.ds(lg * _L, _L)])

        @plsc.parallel_loop(0, _W, unroll=8)
        def _(l):
            off_l = (l // _L) * 1088 + (l % _L) + v17
            for c in range(_D // _L):
                vals = plsc.load_gather(st_v, [off_l + c * (_L * 17)])
                pos = 4 * l + c
                rb_v[p, pos // 8, pl.ds((pos % 8) * _L, _L)] = vals * _SCALE

    start_in(0, 0)

    @pl.loop(0, _NK - 1, step=2)
    def _(k2):
        for b in range(2):
            k = k2 + b
            wait_in(k, b)
            start_in(k + 1, 1 - b)

            @pl.when(k >= 2)
            def _():
                wait_out(k - 2, b)

            compute(b)
            start_out(k, b)

    wait_in(_NK - 1, 0)
    wait_out(_NK - 3, 0)
    compute(0)
    start_out(_NK - 1, 0)
    wait_out(_NK - 2, 1)
    wait_out(_NK - 1, 0)

    @pl.when(wid == _NW - 1)
    def _():
        pltpu.async_copy(tail_hbm, tv_v, sem_in.at[0]).wait()

        @plsc.parallel_loop(0, 32, unroll=4)
        def _(q):
            for h in range(8):
                tv_v[q, pl.ds(h * _L, _L)] = tv_v[q, pl.ds(h * _L, _L)] * _SCALE

        pltpu.async_copy(
            tv_v, out_hbm.at[pl.ds(_NB * _D, 32), :], sem_in.at[0]).wait()


def _emb_pipeline(table_hbm, idx_hbm, out_hbm, iv_v, rows_v, st_v, ob_v,
                  sem_i, sem_g, sem_o, *, num_windows, n_bh):
    wid = lax.axis_index("s") * 2 + lax.axis_index("c")
    lanes = lax.iota(jnp.int32, _L)
    v17 = lanes * 17
    n_per = num_windows // _NW
    w0 = wid * n_per

    def start_idx(k, p):
        pltpu.async_copy(
            idx_hbm.at[0, pl.ds((w0 + k) * _W, _W)], iv_v.at[p], sem_i.at[p])

    def wait_idx(k, p):
        pltpu.make_async_copy(
            idx_hbm.at[0, pl.ds((w0 + k) * _W, _W)], iv_v.at[p],
            sem_i.at[p]).wait()

    def start_gather(p):
        pltpu.async_copy(table_hbm.at[iv_v.at[p]], rows_v.at[p], sem_g.at[p])

    def wait_gather(p):
        pltpu.make_async_copy(
            table_hbm.at[iv_v.at[p]], rows_v.at[p], sem_g.at[p]).wait()

    def out_dst(k):
        w = w0 + k
        return out_hbm.at[w // n_bh, :, w % n_bh, :, :]

    def start_out(k, p):
        pltpu.async_copy(ob_v.at[p], out_dst(k), sem_o.at[p])

    def wait_out(k, p):
        pltpu.make_async_copy(ob_v.at[p], out_dst(k), sem_o.at[p]).wait()

    def compute(p):
        @plsc.parallel_loop(0, _W, unroll=8)
        def _(b):
            for c in range(_D // _L):
                st_v[pl.ds(c * 2176 + b * 17, _L)] = (
                    rows_v[p, b, pl.ds(c * _L, _L)])

        @plsc.parallel_loop(0, _D, unroll=8)
        def _(d):
            off_d = (d // _L) * 2176 + (d % _L) + v17
            for bg in range(_W // _L):
                vals = plsc.load_gather(st_v, [off_d + bg * (_L * 17)])
                ob_v[p, d // 8, d % 8, pl.ds(bg * _L, _L)] = vals

    start_idx(0, 0)
    wait_idx(0, 0)
    start_gather(0)
    start_idx(1, 1)

    @pl.loop(0, n_per - 2, step=2)
    def _(k2):
        for b in range(2):
            k = k2 + b
            wait_idx(k + 1, 1 - b)
            start_gather(1 - b)
            wait_gather(b)
            start_idx(k + 2, b)

            @pl.when(k >= 2)
            def _():
                wait_out(k - 2, b)

            compute(b)
            start_out(k, b)

    wait_idx(n_per - 1, 1)
    start_gather(1)
    wait_gather(0)
    wait_out(n_per - 4, 0)
    compute(0)
    start_out(n_per - 2, 0)
    wait_gather(1)
    wait_out(n_per - 3, 1)
    compute(1)
    start_out(n_per - 1, 1)
    wait_out(n_per - 2, 0)
    wait_out(n_per - 1, 1)


def kernel(inputs, table):
    batch, seq = inputs.shape
    n = batch * seq
    vocab = table.shape[0]
    n_bh = batch // _W
    idx = inputs.astype(jnp.int32).T.reshape(1, n)
    num_windows = n // _W

    mesh = plsc.VectorSubcoreMesh(core_axis_name="c", subcore_axis_name="s")

    @pl.kernel(
        out_type=jax.ShapeDtypeStruct((vocab // 2, 2 * _D), table.dtype),
        mesh=mesh,
        compiler_params=pltpu.CompilerParams(
            use_tc_tiling_on_sc=True, needs_layout_passes=False),
        scratch_types=[
            pltpu.VMEM((2, _D, _W), jnp.float32),
            pltpu.VMEM((_W // _L * _D * 17,), jnp.float32),
            pltpu.VMEM((2, _D, _W), jnp.float32),
            pltpu.VMEM((32, 2 * _D), jnp.float32),
            pltpu.SemaphoreType.DMA((2,)),
            pltpu.SemaphoreType.DMA((2,)),
        ],
    )
    def scale_table(tT_hbm, tail_hbm, out_hbm, in_v, st_v, rb_v, tv_v,
                    sem_in, sem_out):
        _scale_table(tT_hbm, tail_hbm, out_hbm, in_v, st_v, rb_v, tv_v,
                     sem_in, sem_out)

    t2 = scale_table(table.T, table[_NB * _W:].reshape(32, 2 * _D))
    tlin = t2.reshape(vocab, _D)

    @pl.kernel(
        out_type=jax.ShapeDtypeStruct((seq, 8, n_bh, 8, _W), table.dtype),
        mesh=mesh,
        compiler_params=pltpu.CompilerParams(
            use_tc_tiling_on_sc=False, needs_layout_passes=False),
        scratch_types=[
            pltpu.VMEM((2, _W), jnp.int32),
            pltpu.VMEM((2, _W, _D), jnp.float32),
            pltpu.VMEM((_D // _L * _W * 17,), jnp.float32),
            pltpu.VMEM((2, 8, 8, _W), jnp.float32),
            pltpu.SemaphoreType.DMA((2,)),
            pltpu.SemaphoreType.DMA((2,)),
            pltpu.SemaphoreType.DMA((2,)),
        ],
    )
    def emb(table_hbm, idx_hbm, out_hbm, iv_v, rows_v, st_v, ob_v,
            sem_i, sem_g, sem_o):
        _emb_pipeline(table_hbm, idx_hbm, out_hbm, iv_v, rows_v, st_v, ob_v,
                      sem_i, sem_g, sem_o,
                      num_windows=num_windows, n_bh=n_bh)

    out5d = emb(tlin, idx)
    return out5d.transpose(2, 4, 0, 1, 3).reshape(batch, seq, _D)


# speedup vs baseline: 1.0056x; 1.0056x over previous
import jax
import jax.numpy as jnp
from jax import lax
from jax.experimental import pallas as pl
from jax.experimental.pallas import tpu as pltpu
from jax.experimental.pallas import tpu_sc as plsc

_D = 64
_SCALE = 8.0
_W = 128
_L = 16
_NW = 32
_NB = 7812
_NK = 245


def _scale_table(tT_hbm, tail_hbm, out_hbm, in_v, st_v, rb_v, tv_v,
                 sem_in, sem_out):
    wid = lax.axis_index("s") * 2 + lax.axis_index("c")
    lanes = lax.iota(jnp.int32, _L)
    base = wid * (_NB // _NW) + jnp.minimum(wid, _NB % _NW)

    def band(k):
        return jnp.minimum(base + k, _NB - 1)

    def start_in(k, p):
        pltpu.async_copy(
            tT_hbm.at[:, pl.ds(band(k) * _W, _W)], in_v.at[p], sem_in.at[p])

    def wait_in(k, p):
        pltpu.make_async_copy(
            tT_hbm.at[:, pl.ds(band(k) * _W, _W)], in_v.at[p],
            sem_in.at[p]).wait()

    def start_out(k, p):
        pltpu.async_copy(
            rb_v.at[p], out_hbm.at[pl.ds(band(k) * _D, _D), :], sem_out.at[p])

    def wait_out(k, p):
        pltpu.make_async_copy(
            rb_v.at[p], out_hbm.at[pl.ds(band(k) * _D, _D), :],
            sem_out.at[p]).wait()

    v17 = lanes * 17

    def compute(p):
        @plsc.parallel_loop(0, _D, unroll=4)
        def _(d):
            for lg in range(_W // _L):
                st_v[pl.ds(lg * 1088 + d * 17, _L)] = (
                    in_v[p, d, pl.ds(lg * _L, _L)])

        @plsc.parallel_loop(0, _W, unroll=4)
        def _(l):
            off_l = (l // _L) * 1088 + (l % _L) + v17
            for c in range(_D // _L):
                vals = plsc.load_gather(st_v, [off_l + c * (_L * 17)])
                pos = 4 * l + c
                rb_v[p, pos // 8, pl.ds((pos % 8) * _L, _L)] = vals * _SCALE

    start_in(0, 0)

    @pl.loop(0, _NK - 1, step=2)
    def _(k2):
        for b in range(2):
            k = k2 + b
            wait_in(k, b)
            start_in(k + 1, 1 - b)

            @pl.when(k >= 2)
            def _():
                wait_out(k - 2, b)

            compute(b)
            start_out(k, b)

    wait_in(_NK - 1, 0)
    wait_out(_NK - 3, 0)
    compute(0)
    start_out(_NK - 1, 0)
    wait_out(_NK - 2, 1)
    wait_out(_NK - 1, 0)

    @pl.when(wid == _NW - 1)
    def _():
        pltpu.async_copy(tail_hbm, tv_v, sem_in.at[0]).wait()

        @plsc.parallel_loop(0, 32, unroll=4)
        def _(q):
            for h in range(8):
                tv_v[q, pl.ds(h * _L, _L)] = tv_v[q, pl.ds(h * _L, _L)] * _SCALE

        pltpu.async_copy(
            tv_v, out_hbm.at[pl.ds(_NB * _D, 32), :], sem_in.at[0]).wait()


def _emb_pipeline(table_hbm, idx_hbm, out_hbm, iv_v, rows_v, st_v, ob_v,
                  sem_i, sem_g, sem_o, *, num_windows, n_bh):
    wid = lax.axis_index("s") * 2 + lax.axis_index("c")
    lanes = lax.iota(jnp.int32, _L)
    v17 = lanes * 17
    n_per = num_windows // _NW
    w0 = wid * n_per

    def start_idx(k, p):
        pltpu.async_copy(
            idx_hbm.at[0, pl.ds((w0 + k) * _W, _W)], iv_v.at[p], sem_i.at[p])

    def wait_idx(k, p):
        pltpu.make_async_copy(
            idx_hbm.at[0, pl.ds((w0 + k) * _W, _W)], iv_v.at[p],
            sem_i.at[p]).wait()

    def start_gather(p):
        pltpu.async_copy(table_hbm.at[iv_v.at[p]], rows_v.at[p], sem_g.at[p])

    def wait_gather(p):
        pltpu.make_async_copy(
            table_hbm.at[iv_v.at[p]], rows_v.at[p], sem_g.at[p]).wait()

    def out_dst(k):
        w = w0 + k
        return out_hbm.at[w // n_bh, :, w % n_bh, :, :]

    def start_out(k, p):
        pltpu.async_copy(ob_v.at[p], out_dst(k), sem_o.at[p])

    def wait_out(k, p):
        pltpu.make_async_copy(ob_v.at[p], out_dst(k), sem_o.at[p]).wait()

    def compute(p):
        @plsc.parallel_loop(0, _W, unroll=4)
        def _(b):
            for c in range(_D // _L):
                st_v[pl.ds(c * 2176 + b * 17, _L)] = (
                    rows_v[p, b, pl.ds(c * _L, _L)])

        @plsc.parallel_loop(0, _D, unroll=4)
        def _(d):
            off_d = (d // _L) * 2176 + (d % _L) + v17
            for bg in range(_W // _L):
                vals = plsc.load_gather(st_v, [off_d + bg * (_L * 17)])
                ob_v[p, d // 8, d % 8, pl.ds(bg * _L, _L)] = vals

    start_idx(0, 0)
    wait_idx(0, 0)
    start_gather(0)
    start_idx(1, 1)

    @pl.loop(0, n_per - 2, step=2)
    def _(k2):
        for b in range(2):
            k = k2 + b
            wait_idx(k + 1, 1 - b)
            start_gather(1 - b)
            wait_gather(b)
            start_idx(k + 2, b)

            @pl.when(k >= 2)
            def _():
                wait_out(k - 2, b)

            compute(b)
            start_out(k, b)

    wait_idx(n_per - 1, 1)
    start_gather(1)
    wait_gather(0)
    wait_out(n_per - 4, 0)
    compute(0)
    start_out(n_per - 2, 0)
    wait_gather(1)
    wait_out(n_per - 3, 1)
    compute(1)
    start_out(n_per - 1, 1)
    wait_out(n_per - 2, 0)
    wait_out(n_per - 1, 1)


def kernel(inputs, table):
    batch, seq = inputs.shape
    n = batch * seq
    vocab = table.shape[0]
    n_bh = batch // _W
    idx = inputs.astype(jnp.int32).T.reshape(1, n)
    num_windows = n // _W

    mesh = plsc.VectorSubcoreMesh(core_axis_name="c", subcore_axis_name="s")

    @pl.kernel(
        out_type=jax.ShapeDtypeStruct((vocab // 2, 2 * _D), table.dtype),
        mesh=mesh,
        compiler_params=pltpu.CompilerParams(
            use_tc_tiling_on_sc=True, needs_layout_passes=False),
        scratch_types=[
            pltpu.VMEM((2, _D, _W), jnp.float32),
            pltpu.VMEM((_W // _L * _D * 17,), jnp.float32),
            pltpu.VMEM((2, _D, _W), jnp.float32),
            pltpu.VMEM((32, 2 * _D), jnp.float32),
            pltpu.SemaphoreType.DMA((2,)),
            pltpu.SemaphoreType.DMA((2,)),
        ],
    )
    def scale_table(tT_hbm, tail_hbm, out_hbm, in_v, st_v, rb_v, tv_v,
                    sem_in, sem_out):
        _scale_table(tT_hbm, tail_hbm, out_hbm, in_v, st_v, rb_v, tv_v,
                     sem_in, sem_out)

    t2 = scale_table(table.T, table[_NB * _W:].reshape(32, 2 * _D))
    tlin = t2.reshape(vocab, _D)

    @pl.kernel(
        out_type=jax.ShapeDtypeStruct((seq, 8, n_bh, 8, _W), table.dtype),
        mesh=mesh,
        compiler_params=pltpu.CompilerParams(
            use_tc_tiling_on_sc=False, needs_layout_passes=False),
        scratch_types=[
            pltpu.VMEM((2, _W), jnp.int32),
            pltpu.VMEM((2, _W, _D), jnp.float32),
            pltpu.VMEM((_D // _L * _W * 17,), jnp.float32),
            pltpu.VMEM((2, 8, 8, _W), jnp.float32),
            pltpu.SemaphoreType.DMA((2,)),
            pltpu.SemaphoreType.DMA((2,)),
            pltpu.SemaphoreType.DMA((2,)),
        ],
    )
    def emb(table_hbm, idx_hbm, out_hbm, iv_v, rows_v, st_v, ob_v,
            sem_i, sem_g, sem_o):
        _emb_pipeline(table_hbm, idx_hbm, out_hbm, iv_v, rows_v, st_v, ob_v,
                      sem_i, sem_g, sem_o,
                      num_windows=num_windows, n_bh=n_bh)

    out5d = emb(tlin, idx)
    return out5d.transpose(2, 4, 0, 1, 3).reshape(batch, seq, _D)
